# baseline (speedup 1.0000x reference)
import jax
import jax.numpy as jnp
from jax import lax
from jax.experimental import pallas as pl
from jax.experimental.pallas import tpu as pltpu
from jax.experimental.pallas import tpu_sc as plsc

_NC = 2
_NS = 16
_NW = _NC * _NS
_L = 16
_NB = 4


def _make_body(batch, seq, embed, bc):
  n_tiles = seq // 8
  ngrp = bc // _L

  def body(xt_hbm, t2_hbm, out_hbm, xi, pidx, hbuf, gbufs, wbufs,
           gsems, wsems):
    wid = lax.axis_index("s") * _NC + lax.axis_index("c")
    b0 = wid * bc
    lane = lax.iota(jnp.int32, _L)

    def load_xtile(t, slot):
      pltpu.sync_copy(xt_hbm.at[pl.ds(t * 8, 8), pl.ds(b0, bc)],
                      xi.at[slot])

    def prep_idx(tp, rp, k):
      for g in range(ngrp):
        v = xi[tp, rp, pl.ds(g * _L, _L)]
        pidx[k][pl.ds(g * _L, _L)] = v >> 1
        hbuf[k][pl.ds(g * _L, _L)] = v & 1

    def start_gather(k):
      pltpu.async_copy(t2_hbm.at[pidx[k]], gbufs[k], gsems[k])

    def wait_gather(k):
      pltpu.make_async_copy(t2_hbm.at[pidx[k]], gbufs[k], gsems[k]).wait()

    def select_transpose(k):
      def g_step(g, carry):
        h16 = hbuf[k][pl.ds(g * _L, _L)]
        row16 = g * _L + lane
        col_base = h16 * embed

        def e_step(eb, carry2):
          for j in range(8):
            e = eb * 8 + j
            val = plsc.load_gather(gbufs[k], [row16, col_base + e])
            wbufs[k][e, pl.ds(g * _L, _L)] = val
          return carry2

        lax.fori_loop(0, embed // 8, e_step, None)
        return carry

      lax.fori_loop(0, ngrp, g_step, None)

    def write_desc(s, k):
      return (wbufs[k], out_hbm.at[s, :, pl.ds(b0, bc)], wsems[k])

    def fire_write(s, k):
      src, dst, sem = write_desc(s, k)
      pltpu.async_copy(src, dst, sem)

    def wait_write(s, k):
      src, dst, sem = write_desc(s, k)
      pltpu.make_async_copy(src, dst, sem).wait()

    load_xtile(0, 0)
    for u in range(_NB):
      prep_idx(0, u, u)
      start_gather(u)

    def t_step(t, carry):
      for r in range(8):
        u = t * 8 + r
        k = r % _NB
        wait_gather(k)
        if r >= _NB:
          wait_write(u - _NB, k)
        else:
          @pl.when(t > 0)
          def _():
            wait_write(u - _NB, k)
        select_transpose(k)
        fire_write(u, k)
        if r == 4:
          @pl.when(t < n_tiles - 1)
          def _():
            load_xtile(t + 1, (t + 1) % 2)
        rp = (r + _NB) % 8
        if r < 8 - _NB:
          prep_idx(t % 2, rp, k)
          start_gather(k)
        else:
          @pl.when(t < n_tiles - 1)
          def _():
            prep_idx((t + 1) % 2, rp, k)
            start_gather(k)
      return carry

    lax.fori_loop(0, n_tiles, t_step, None)

    for j in range(_NB):
      wait_write(seq - _NB + j, (8 - _NB + j) % _NB)

  return body


@jax.jit
def kernel(x, table):
  batch, seq = x.shape
  vocab, embed = table.shape
  bc = batch // _NW
  x_t = jnp.swapaxes(x, 0, 1)
  t2 = table.reshape(vocab // 2, 2 * embed)

  mesh = plsc.VectorSubcoreMesh(core_axis_name="c", subcore_axis_name="s")
  out3 = pl.kernel(
      _make_body(batch, seq, embed, bc),
      out_type=jax.ShapeDtypeStruct((seq, embed, batch), jnp.float32),
      mesh=mesh,
      compiler_params=pltpu.CompilerParams(
          needs_layout_passes=False, use_tc_tiling_on_sc=True),
      scratch_types=[
          pltpu.VMEM((2, 8, bc), jnp.int32),
          [pltpu.VMEM((bc,), jnp.int32) for _ in range(_NB)],
          [pltpu.VMEM((bc,), jnp.int32) for _ in range(_NB)],
          [pltpu.VMEM((bc, 2 * embed), jnp.float32) for _ in range(_NB)],
          [pltpu.VMEM((embed, bc), jnp.float32) for _ in range(_NB)],
          [pltpu.SemaphoreType.DMA for _ in range(_NB)],
          [pltpu.SemaphoreType.DMA for _ in range(_NB)],
      ],
  )(x_t, t2)
  return jnp.swapaxes(out3, 1, 2)


# speedup vs baseline: 1.5457x; 1.5457x over previous
import jax
import jax.numpy as jnp
from jax import lax
from jax.experimental import pallas as pl
from jax.experimental.pallas import tpu as pltpu
from jax.experimental.pallas import tpu_sc as plsc

_NC = 2
_NS = 16
_NW = _NC * _NS
_G = 2
_NBUF = 4


def _make_body(batch, seq, embed, bc):
  n_stages = seq // _G

  def body(xt_hbm, table_hbm, out_hbm, idx_bufs, row_bufs, gsems, wsems):
    wid = lax.axis_index("s") * _NC + lax.axis_index("c")
    b0 = wid * bc

    def start_gather(t, k):
      s0 = t * _G
      for i in range(_G):
        pltpu.sync_copy(xt_hbm.at[s0 + i, pl.ds(b0, bc)],
                        idx_bufs[k].at[pl.ds(i * bc, bc)])
      pltpu.async_copy(table_hbm.at[idx_bufs[k]], row_bufs[k], gsems[k])

    def wait_gather(k):
      pltpu.make_async_copy(
          table_hbm.at[idx_bufs[k]], row_bufs[k], gsems[k]).wait()

    def write_descs(t, k):
      for i in range(_G):
        yield (row_bufs[k].at[pl.ds(i * bc, bc)],
               out_hbm.at[pl.ds((t * _G + i) * batch + b0, bc)], wsems[k])

    def fire_writes(t, k):
      for src, dst, sem in write_descs(t, k):
        pltpu.async_copy(src, dst, sem)

    def wait_writes(t, k):
      for src, dst, sem in write_descs(t, k):
        pltpu.make_async_copy(src, dst, sem).wait()

    for k in range(_NBUF):
      start_gather(k, k)

    def step(g, carry):
      for k in range(_NBUF):
        t = g * _NBUF + k
        kprev = (k - 1) % _NBUF
        wait_gather(k)
        fire_writes(t, k)

        @pl.when((t >= 1) & (t + _NBUF - 1 < n_stages))
        def _():
          wait_writes(t - 1, kprev)
          start_gather(t - 1 + _NBUF, kprev)
      return carry

    lax.fori_loop(0, n_stages // _NBUF, step, None)

    for k in range(_NBUF):
      wait_writes(n_stages - _NBUF + k, k)

  return body


@jax.jit
def kernel(x, table):
  batch, seq = x.shape
  _, embed = table.shape
  bc = batch // _NW
  x_t = jnp.swapaxes(x, 0, 1)

  mesh = plsc.VectorSubcoreMesh(core_axis_name="c", subcore_axis_name="s")
  out = pl.kernel(
      _make_body(batch, seq, embed, bc),
      out_type=jax.ShapeDtypeStruct((seq * batch, embed), jnp.float32),
      mesh=mesh,
      compiler_params=pltpu.CompilerParams(
          needs_layout_passes=False, use_tc_tiling_on_sc=False),
      scratch_types=[
          [pltpu.VMEM((_G * bc,), jnp.int32) for _ in range(_NBUF)],
          [pltpu.VMEM((_G * bc, embed), jnp.float32) for _ in range(_NBUF)],
          [pltpu.SemaphoreType.DMA for _ in range(_NBUF)],
          [pltpu.SemaphoreType.DMA for _ in range(_NBUF)],
      ],
  )(x_t, table)
  return out.reshape(seq, batch, embed)
